# Initial kernel scaffold; baseline (speedup 1.0000x reference)
import functools

import jax
import jax.numpy as jnp
from jax import lax
from jax.experimental import pallas as pl
from jax.experimental.pallas import tpu as pltpu
from jax.experimental.pallas import tpu_sc as plsc

N = 50000
E = 1600000
H = 32
NUM_CLASSES = 128
B = 32

NC = 2
NS = 16
NTILES = NC * NS
EW = 128
NWIN_E = E // EW
NPAD = 51200
TPN = NPAD // NS
NODE_PAD = 50048
NWIN_N = NODE_PAD // 128

_mesh = plsc.VectorSubcoreMesh(core_axis_name="c", subcore_axis_name="s")


def _zero_rows_buf(buf):
  z16 = jnp.zeros((16,), jnp.float32)

  @pl.loop(0, 128)
  def _(r):
    buf[r, pl.ds(0, 16)] = z16
    buf[r, pl.ds(16, 16)] = z16


def _sc_prep_body(dst_hbm, xt_hbm, emb_hbm, degp_hbm, h0_hbm,
                  dstb, onesb, zdeg, idxb, rows, acc, shared_deg, sem):
  cid = lax.axis_index("c")
  sid = lax.axis_index("s")
  wid = sid * NC + cid

  @pl.loop(0, TPN // 16)
  def _(k):
    zdeg[pl.ds(k * 16, 16)] = jnp.zeros((16,), jnp.float32)

  pltpu.sync_copy(zdeg, shared_deg.at[pl.ds(sid * TPN, TPN)])

  for k in range(8):
    onesb[pl.ds(k * 16, 16)] = jnp.full((16,), 1.0, jnp.float32)

  plsc.subcore_barrier()

  @pl.loop(wid, NWIN_E, step=NTILES)
  def _(w):
    pltpu.sync_copy(dst_hbm.at[pl.ds(w, 1)], dstb)
    pltpu.sync_copy(onesb, shared_deg.at[dstb.at[0]], add=True)

  plsc.subcore_barrier()

  pltpu.sync_copy(shared_deg.at[pl.ds(sid * TPN, TPN)],
                  degp_hbm.at[cid].at[pl.ds(sid * TPN, TPN)])

  @pl.loop(wid, NWIN_N, step=NTILES)
  def _(w):
    for i in range(9):
      pltpu.sync_copy(xt_hbm.at[i].at[pl.ds(w, 1)], idxb)
      pltpu.async_copy(emb_hbm.at[idxb.at[0]],
                       rows.at[pl.ds(i * 128, 128)], sem).wait()

    @pl.loop(0, 128)
    def _(r):
      for half in range(2):
        cs = pl.ds(half * 16, 16)
        v = rows[r, cs]
        for i in range(1, 9):
          v = v + rows[i * 128 + r, cs]
        acc[r, cs] = v

    pltpu.sync_copy(acc, h0_hbm.at[pl.ds(w * 128, 128)])


def _sc_prep(dst2d, xt9, emb_flat):
  kfn = pl.kernel(
      _sc_prep_body,
      out_type=(
          jax.ShapeDtypeStruct((NC, NPAD), jnp.float32),
          jax.ShapeDtypeStruct((NODE_PAD, H), jnp.float32),
      ),
      mesh=_mesh,
      scratch_types=[
          pltpu.VMEM((1, EW), jnp.int32),
          pltpu.VMEM((EW,), jnp.float32),
          pltpu.VMEM((TPN,), jnp.float32),
          pltpu.VMEM((1, EW), jnp.int32),
          pltpu.VMEM((9 * 128, H), jnp.float32),
          pltpu.VMEM((128, H), jnp.float32),
          pltpu.VMEM_SHARED((NPAD,), jnp.float32),
          pltpu.SemaphoreType.DMA,
      ],
  )
  return kfn(dst2d, xt9, emb_flat)


def _sc_edge_body(g_hbm, src_hbm, dst_hbm, out_hbm,
                  srcb, dstb, rows, zbuf, shared_acc, sem):
  cid = lax.axis_index("c")
  sid = lax.axis_index("s")
  wid = sid * NC + cid

  _zero_rows_buf(zbuf)

  @pl.loop(0, TPN // 128)
  def _(k):
    pltpu.sync_copy(zbuf, shared_acc.at[pl.ds(sid * TPN + k * 128, 128)])

  plsc.subcore_barrier()

  @pl.loop(wid, NWIN_E, step=NTILES)
  def _(w):
    pltpu.sync_copy(src_hbm.at[pl.ds(w, 1)], srcb)
    pltpu.sync_copy(dst_hbm.at[pl.ds(w, 1)], dstb)
    pltpu.async_copy(g_hbm.at[srcb.at[0]], rows, sem).wait()
    pltpu.sync_copy(rows, shared_acc.at[dstb.at[0]], add=True)

  plsc.subcore_barrier()

  @pl.loop(0, TPN // 128)
  def _(k):
    pltpu.sync_copy(shared_acc.at[pl.ds(sid * TPN + k * 128, 128)],
                    out_hbm.at[cid].at[pl.ds(sid * TPN + k * 128, 128)])


def _sc_edge(g, src2d, dst2d):
  kfn = pl.kernel(
      _sc_edge_body,
      out_type=jax.ShapeDtypeStruct((NC, NPAD, H), jnp.float32),
      mesh=_mesh,
      scratch_types=[
          pltpu.VMEM((1, EW), jnp.int32),
          pltpu.VMEM((1, EW), jnp.int32),
          pltpu.VMEM((EW, H), jnp.float32),
          pltpu.VMEM((128, H), jnp.float32),
          pltpu.VMEM_SHARED((NPAD, H), jnp.float32),
          pltpu.SemaphoreType.DMA,
      ],
  )
  return kfn(g, src2d, dst2d)


_DOT = functools.partial(
    lax.dot_general,
    precision=lax.Precision.HIGHEST,
    preferred_element_type=jnp.float32,
)


def _mm(a, b):
  return _DOT(a, b, dimension_numbers=(((1,), (0,)), ((), ())))


RB = 2000
GRID_N = N // RB


def _t1_body(h0_ref, dga_ref, dgb_ref, w1_ref, g1_ref, dinv_ref):
  deg = dga_ref[...] + dgb_ref[...] + 1.0
  dinv = lax.rsqrt(deg)
  dinv_ref[...] = dinv
  g1_ref[...] = _mm(h0_ref[...], w1_ref[...]) * dinv


def _t1(h0, dga, dgb, w1):
  return pl.pallas_call(
      _t1_body,
      grid=(GRID_N,),
      in_specs=[
          pl.BlockSpec((RB, H), lambda i: (i, 0)),
          pl.BlockSpec((RB, 1), lambda i: (i, 0)),
          pl.BlockSpec((RB, 1), lambda i: (i, 0)),
          pl.BlockSpec((H, H), lambda i: (0, 0)),
      ],
      out_specs=[
          pl.BlockSpec((RB, H), lambda i: (i, 0)),
          pl.BlockSpec((RB, 1), lambda i: (i, 0)),
      ],
      out_shape=[
          jax.ShapeDtypeStruct((N, H), jnp.float32),
          jax.ShapeDtypeStruct((N, 1), jnp.float32),
      ],
  )(h0, dga, dgb, w1)


def _t2_body(s1a_ref, s1b_ref, g1_ref, dinv_ref, b1_ref, w2_ref, g2_ref):
  dinv = dinv_ref[...]
  h1 = dinv * (s1a_ref[...] + s1b_ref[...] + g1_ref[...]) + b1_ref[...]
  h1 = jnp.maximum(h1, 0.0)
  g2_ref[...] = _mm(h1, w2_ref[...]) * dinv


def _t2(s1a, s1b, g1, dinv, b1, w2):
  return pl.pallas_call(
      _t2_body,
      grid=(GRID_N,),
      in_specs=[
          pl.BlockSpec((RB, H), lambda i: (i, 0)),
          pl.BlockSpec((RB, H), lambda i: (i, 0)),
          pl.BlockSpec((RB, H), lambda i: (i, 0)),
          pl.BlockSpec((RB, 1), lambda i: (i, 0)),
          pl.BlockSpec((1, H), lambda i: (0, 0)),
          pl.BlockSpec((H, H), lambda i: (0, 0)),
      ],
      out_specs=pl.BlockSpec((RB, H), lambda i: (i, 0)),
      out_shape=jax.ShapeDtypeStruct((N, H), jnp.float32),
  )(s1a, s1b, g1, dinv, b1, w2)


def _t3_body(s2a_ref, s2b_ref, g2_ref, dinv_ref, b2_ref, batch_ref,
             wout_ref, bout_ref, out_ref, sums_ref, cnt_ref):
  i = pl.program_id(0)

  @pl.when(i == 0)
  def _():
    sums_ref[...] = jnp.zeros_like(sums_ref)
    cnt_ref[...] = jnp.zeros_like(cnt_ref)

  dinv = dinv_ref[...]
  h2 = dinv * (s2a_ref[...] + s2b_ref[...] + g2_ref[...]) + b2_ref[...]
  seg = lax.broadcasted_iota(jnp.int32, (B, RB), 0)
  mask = (seg == batch_ref[...]).astype(jnp.float32)
  sums_ref[...] += _mm(mask, h2)
  cnt_ref[...] += jnp.sum(mask, axis=1, keepdims=True)

  @pl.when(i == GRID_N - 1)
  def _():
    pooled = sums_ref[...] / jnp.maximum(cnt_ref[...], 1.0)
    out_ref[...] = _mm(pooled, wout_ref[...]) + bout_ref[...]


def _t3(s2a, s2b, g2, dinv, b2, batch2d, wout, bout):
  return pl.pallas_call(
      _t3_body,
      grid=(GRID_N,),
      in_specs=[
          pl.BlockSpec((RB, H), lambda i: (i, 0)),
          pl.BlockSpec((RB, H), lambda i: (i, 0)),
          pl.BlockSpec((RB, H), lambda i: (i, 0)),
          pl.BlockSpec((RB, 1), lambda i: (i, 0)),
          pl.BlockSpec((1, H), lambda i: (0, 0)),
          pl.BlockSpec((1, RB), lambda i: (i, 0)),
          pl.BlockSpec((H, NUM_CLASSES), lambda i: (0, 0)),
          pl.BlockSpec((1, NUM_CLASSES), lambda i: (0, 0)),
      ],
      out_specs=pl.BlockSpec((B, NUM_CLASSES), lambda i: (0, 0)),
      out_shape=jax.ShapeDtypeStruct((B, NUM_CLASSES), jnp.float32),
      scratch_shapes=[
          pltpu.VMEM((B, H), jnp.float32),
          pltpu.VMEM((B, 1), jnp.float32),
      ],
  )(s2a, s2b, g2, dinv, b2, batch2d, wout, bout)


@jax.jit
def kernel(x, edge_index, batch, emb, W1, b1, W2, b2, Wout, bout):
  x = x.astype(jnp.int32)
  edge_index = edge_index.astype(jnp.int32)
  batch = batch.astype(jnp.int32)

  xi = x + (jnp.arange(9, dtype=jnp.int32) * 100)[None, :]
  xt = jnp.zeros((9, NODE_PAD), jnp.int32).at[:, :N].set(xi.T)
  xt9 = xt.reshape(9, NWIN_N, 128)
  emb_flat = emb.reshape(9 * 100, H)

  src2d = edge_index[0].reshape(NWIN_E, EW)
  dst2d = edge_index[1].reshape(NWIN_E, EW)

  degp, h0p = _sc_prep(dst2d, xt9, emb_flat)
  h0 = h0p[:N]
  dga = degp[0, :N, None]
  dgb = degp[1, :N, None]

  g1, dinv = _t1(h0, dga, dgb, W1)

  s1 = _sc_edge(g1, src2d, dst2d)
  g2 = _t2(s1[0, :N], s1[1, :N], g1, dinv, b1.reshape(1, H), W2)

  s2 = _sc_edge(g2, src2d, dst2d)
  out = _t3(s2[0, :N], s2[1, :N], g2, dinv, b2.reshape(1, H),
            batch.reshape(GRID_N, RB), Wout, bout.reshape(1, NUM_CLASSES))
  return out


# speedup vs baseline: 16.1488x; 16.1488x over previous
import functools

import jax
import jax.numpy as jnp
from jax import lax
from jax.experimental import pallas as pl
from jax.experimental.pallas import tpu as pltpu
from jax.experimental.pallas import tpu_sc as plsc

N = 50000
E = 1600000
H = 32
NUM_CLASSES = 128
B = 32

NC = 2
NS = 16
NTILES = NC * NS
EW = 128
NWIN_E = E // EW
NPAD = 51200
TPN = NPAD // NS
NODE_PAD = 50048
NWIN_N = NODE_PAD // 128

_mesh = plsc.VectorSubcoreMesh(core_axis_name="c", subcore_axis_name="s")
_SC_PARAMS = pltpu.CompilerParams(use_tc_tiling_on_sc=False)


def _zero_rows_buf(buf):
  z16 = jnp.zeros((16,), jnp.float32)

  @pl.loop(0, 128)
  def _(r):
    buf[r, pl.ds(0, 16)] = z16
    buf[r, pl.ds(16, 16)] = z16


def _sc_prep_body(dst_hbm, xt_hbm, emb_hbm, degp_hbm, h0_hbm,
                  dstb, onesb, zdeg, idxb, rows, acc, shared_deg, sem):
  cid = lax.axis_index("c")
  sid = lax.axis_index("s")
  wid = sid * NC + cid

  @pl.loop(0, TPN // 16)
  def _(k):
    zdeg[pl.ds(k * 16, 16)] = jnp.zeros((16,), jnp.float32)

  pltpu.sync_copy(zdeg, shared_deg.at[pl.ds(sid * TPN, TPN)])

  for k in range(8):
    onesb[pl.ds(k * 16, 16)] = jnp.full((16,), 1.0, jnp.float32)

  plsc.subcore_barrier()

  @pl.loop(wid, NWIN_E, step=NTILES)
  def _(w):
    pltpu.sync_copy(dst_hbm.at[pl.ds(w, 1)], dstb)
    pltpu.sync_copy(onesb, shared_deg.at[dstb.at[0]], add=True)

  plsc.subcore_barrier()

  pltpu.sync_copy(shared_deg.at[pl.ds(sid * TPN, TPN)],
                  degp_hbm.at[cid].at[pl.ds(sid * TPN, TPN)])

  @pl.loop(wid, NWIN_N, step=NTILES)
  def _(w):
    for i in range(9):
      pltpu.sync_copy(xt_hbm.at[i].at[pl.ds(w, 1)], idxb)
      pltpu.async_copy(emb_hbm.at[idxb.at[0]],
                       rows.at[pl.ds(i * 128, 128)], sem).wait()

    @pl.loop(0, 128)
    def _(r):
      for half in range(2):
        cs = pl.ds(half * 16, 16)
        v = rows[r, cs]
        for i in range(1, 9):
          v = v + rows[i * 128 + r, cs]
        acc[r, cs] = v

    pltpu.sync_copy(acc, h0_hbm.at[pl.ds(w * 128, 128)])


def _sc_prep(dst2d, xt9, emb_flat):
  kfn = pl.kernel(
      _sc_prep_body,
      out_type=(
          jax.ShapeDtypeStruct((NC, NPAD), jnp.float32),
          jax.ShapeDtypeStruct((NODE_PAD, H), jnp.float32),
      ),
      mesh=_mesh,
      scratch_types=[
          pltpu.VMEM((1, EW), jnp.int32),
          pltpu.VMEM((EW,), jnp.float32),
          pltpu.VMEM((TPN,), jnp.float32),
          pltpu.VMEM((1, EW), jnp.int32),
          pltpu.VMEM((9 * 128, H), jnp.float32),
          pltpu.VMEM((128, H), jnp.float32),
          pltpu.VMEM_SHARED((NPAD,), jnp.float32),
          pltpu.SemaphoreType.DMA,
      ],
      compiler_params=_SC_PARAMS,
  )
  return kfn(dst2d, xt9, emb_flat)


def _sc_edge_body(g_hbm, src_hbm, dst_hbm, out_hbm,
                  srcb, dstb, rows, zbuf, shared_acc, sem):
  cid = lax.axis_index("c")
  sid = lax.axis_index("s")
  wid = sid * NC + cid

  _zero_rows_buf(zbuf)

  @pl.loop(0, TPN // 128)
  def _(k):
    pltpu.sync_copy(zbuf, shared_acc.at[pl.ds(sid * TPN + k * 128, 128)])

  plsc.subcore_barrier()

  @pl.loop(wid, NWIN_E, step=NTILES)
  def _(w):
    pltpu.sync_copy(src_hbm.at[pl.ds(w, 1)], srcb)
    pltpu.sync_copy(dst_hbm.at[pl.ds(w, 1)], dstb)
    pltpu.async_copy(g_hbm.at[srcb.at[0]], rows, sem).wait()
    pltpu.sync_copy(rows, shared_acc.at[dstb.at[0]], add=True)

  plsc.subcore_barrier()

  @pl.loop(0, TPN // 128)
  def _(k):
    pltpu.sync_copy(shared_acc.at[pl.ds(sid * TPN + k * 128, 128)],
                    out_hbm.at[cid].at[pl.ds(sid * TPN + k * 128, 128)])


def _sc_edge(g, src2d, dst2d):
  kfn = pl.kernel(
      _sc_edge_body,
      out_type=jax.ShapeDtypeStruct((NC, NPAD, H), jnp.float32),
      mesh=_mesh,
      scratch_types=[
          pltpu.VMEM((1, EW), jnp.int32),
          pltpu.VMEM((1, EW), jnp.int32),
          pltpu.VMEM((EW, H), jnp.float32),
          pltpu.VMEM((128, H), jnp.float32),
          pltpu.VMEM_SHARED((NPAD, H), jnp.float32),
          pltpu.SemaphoreType.DMA,
      ],
      compiler_params=_SC_PARAMS,
  )
  return kfn(g, src2d, dst2d)


_DOT = functools.partial(
    lax.dot_general,
    precision=lax.Precision.HIGHEST,
    preferred_element_type=jnp.float32,
)


def _mm(a, b):
  return _DOT(a, b, dimension_numbers=(((1,), (0,)), ((), ())))


RB = 2000
GRID_N = N // RB


def _t1_body(h0_ref, dga_ref, dgb_ref, w1_ref, g1_ref, dinv_ref):
  deg = dga_ref[...] + dgb_ref[...] + 1.0
  dinv = lax.rsqrt(deg)
  dinv_ref[...] = dinv
  g1_ref[...] = _mm(h0_ref[...], w1_ref[...]) * dinv


def _t1(h0, dga, dgb, w1):
  return pl.pallas_call(
      _t1_body,
      grid=(GRID_N,),
      in_specs=[
          pl.BlockSpec((RB, H), lambda i: (i, 0)),
          pl.BlockSpec((RB, 1), lambda i: (i, 0)),
          pl.BlockSpec((RB, 1), lambda i: (i, 0)),
          pl.BlockSpec((H, H), lambda i: (0, 0)),
      ],
      out_specs=[
          pl.BlockSpec((RB, H), lambda i: (i, 0)),
          pl.BlockSpec((RB, 1), lambda i: (i, 0)),
      ],
      out_shape=[
          jax.ShapeDtypeStruct((N, H), jnp.float32),
          jax.ShapeDtypeStruct((N, 1), jnp.float32),
      ],
  )(h0, dga, dgb, w1)


def _t2_body(s1a_ref, s1b_ref, g1_ref, dinv_ref, b1_ref, w2_ref, g2_ref):
  dinv = dinv_ref[...]
  h1 = dinv * (s1a_ref[...] + s1b_ref[...] + g1_ref[...]) + b1_ref[...]
  h1 = jnp.maximum(h1, 0.0)
  g2_ref[...] = _mm(h1, w2_ref[...]) * dinv


def _t2(s1a, s1b, g1, dinv, b1, w2):
  return pl.pallas_call(
      _t2_body,
      grid=(GRID_N,),
      in_specs=[
          pl.BlockSpec((RB, H), lambda i: (i, 0)),
          pl.BlockSpec((RB, H), lambda i: (i, 0)),
          pl.BlockSpec((RB, H), lambda i: (i, 0)),
          pl.BlockSpec((RB, 1), lambda i: (i, 0)),
          pl.BlockSpec((1, H), lambda i: (0, 0)),
          pl.BlockSpec((H, H), lambda i: (0, 0)),
      ],
      out_specs=pl.BlockSpec((RB, H), lambda i: (i, 0)),
      out_shape=jax.ShapeDtypeStruct((N, H), jnp.float32),
  )(s1a, s1b, g1, dinv, b1, w2)


def _t3_body(s2a_ref, s2b_ref, g2_ref, dinv_ref, b2_ref, batch_ref,
             wout_ref, bout_ref, out_ref, sums_ref, cnt_ref):
  i = pl.program_id(0)

  @pl.when(i == 0)
  def _():
    sums_ref[...] = jnp.zeros_like(sums_ref)
    cnt_ref[...] = jnp.zeros_like(cnt_ref)

  dinv = dinv_ref[...]
  h2 = dinv * (s2a_ref[...] + s2b_ref[...] + g2_ref[...]) + b2_ref[...]
  seg = lax.broadcasted_iota(jnp.int32, (B, RB), 0)
  mask = (seg == batch_ref[...][0]).astype(jnp.float32)
  sums_ref[...] += _mm(mask, h2)
  cnt_ref[...] += jnp.sum(mask, axis=1, keepdims=True)

  @pl.when(i == GRID_N - 1)
  def _():
    pooled = sums_ref[...] / jnp.maximum(cnt_ref[...], 1.0)
    out_ref[...] = _mm(pooled, wout_ref[...]) + bout_ref[...]


def _t3(s2a, s2b, g2, dinv, b2, batch2d, wout, bout):
  return pl.pallas_call(
      _t3_body,
      grid=(GRID_N,),
      in_specs=[
          pl.BlockSpec((RB, H), lambda i: (i, 0)),
          pl.BlockSpec((RB, H), lambda i: (i, 0)),
          pl.BlockSpec((RB, H), lambda i: (i, 0)),
          pl.BlockSpec((RB, 1), lambda i: (i, 0)),
          pl.BlockSpec((1, H), lambda i: (0, 0)),
          pl.BlockSpec((1, 1, RB), lambda i: (i, 0, 0)),
          pl.BlockSpec((H, NUM_CLASSES), lambda i: (0, 0)),
          pl.BlockSpec((1, NUM_CLASSES), lambda i: (0, 0)),
      ],
      out_specs=pl.BlockSpec((B, NUM_CLASSES), lambda i: (0, 0)),
      out_shape=jax.ShapeDtypeStruct((B, NUM_CLASSES), jnp.float32),
      scratch_shapes=[
          pltpu.VMEM((B, H), jnp.float32),
          pltpu.VMEM((B, 1), jnp.float32),
      ],
  )(s2a, s2b, g2, dinv, b2, batch2d, wout, bout)


@jax.jit
def kernel(x, edge_index, batch, emb, W1, b1, W2, b2, Wout, bout):
  x = x.astype(jnp.int32)
  edge_index = edge_index.astype(jnp.int32)
  batch = batch.astype(jnp.int32)

  xi = x + (jnp.arange(9, dtype=jnp.int32) * 100)[None, :]
  xt = jnp.zeros((9, NODE_PAD), jnp.int32).at[:, :N].set(xi.T)
  xt9 = xt.reshape(9, NWIN_N, 128)
  emb_flat = emb.reshape(9 * 100, H)

  src2d = edge_index[0].reshape(NWIN_E, EW)
  dst2d = edge_index[1].reshape(NWIN_E, EW)

  degp, h0p = _sc_prep(dst2d, xt9, emb_flat)
  h0 = h0p[:N]
  dga = degp[0, :N, None]
  dgb = degp[1, :N, None]

  g1, dinv = _t1(h0, dga, dgb, W1)

  s1 = _sc_edge(g1, src2d, dst2d)
  g2 = _t2(s1[0, :N], s1[1, :N], g1, dinv, b1.reshape(1, H), W2)

  s2 = _sc_edge(g2, src2d, dst2d)
  out = _t3(s2[0, :N], s2[1, :N], g2, dinv, b2.reshape(1, H),
            batch.reshape(GRID_N, 1, RB), Wout, bout.reshape(1, NUM_CLASSES))
  return out


# speedup vs baseline: 30.7237x; 1.9025x over previous
import functools

import jax
import jax.numpy as jnp
from jax import lax
from jax.experimental import pallas as pl
from jax.experimental.pallas import tpu as pltpu
from jax.experimental.pallas import tpu_sc as plsc

N = 50000
E = 1600000
H = 32
NUM_CLASSES = 128
B = 32

NC = 2
NS = 16
NTILES = NC * NS
EW = 128
NWIN_E = E // EW
SUP = 2
NSUP = NWIN_E // SUP
SUPD = 5
NSUPD = NWIN_E // SUPD
NPAD = 51200
TPN = NPAD // NS
NODE_PAD = 50048
NWIN_N = NODE_PAD // 128

_mesh = plsc.VectorSubcoreMesh(core_axis_name="c", subcore_axis_name="s")
_SC_PARAMS = pltpu.CompilerParams(use_tc_tiling_on_sc=False)


ZR = 64


def _zero_rows_buf(buf):
  z16 = jnp.zeros((16,), jnp.float32)

  @pl.loop(0, ZR)
  def _(r):
    buf[r, pl.ds(0, 16)] = z16
    buf[r, pl.ds(16, 16)] = z16


def _sc_prep_body(ei_hbm, xt_hbm, emb_hbm, degp_hbm, h0_hbm,
                  dstb, onesb, zdeg, idxb, rows, acc, shared_deg,
                  sem_d, sem_g, sem_w):
  cid = lax.axis_index("c")
  sid = lax.axis_index("s")
  wid = sid * NC + cid

  @pl.loop(0, TPN // 16)
  def _(k):
    zdeg[pl.ds(k * 16, 16)] = jnp.zeros((16,), jnp.float32)

  pltpu.sync_copy(zdeg, shared_deg.at[pl.ds(sid * TPN, TPN)])

  for k in range(8):
    onesb[pl.ds(k * 16, 16)] = jnp.full((16,), 1.0, jnp.float32)

  plsc.subcore_barrier()

  @pl.loop(wid, NSUPD, step=2 * NTILES)
  def _(t):
    for u in range(2):
      tt = t + u * NTILES

      @pl.when(tt < NSUPD)
      def _():
        @pl.when(tt >= wid + 2 * NTILES)
        def _():
          for j in range(SUPD):
            pltpu.make_async_copy(
                onesb, shared_deg.at[dstb.at[u].at[j]], sem_d.at[u]).wait()

        pltpu.sync_copy(ei_hbm.at[1].at[pl.ds(tt * SUPD, SUPD)], dstb.at[u])
        for j in range(SUPD):
          pltpu.async_copy(
              onesb, shared_deg.at[dstb.at[u].at[j]], sem_d.at[u], add=True)

  for u in range(2):
    for j in range(SUPD):
      pltpu.make_async_copy(
          onesb, shared_deg.at[dstb.at[u].at[j]], sem_d.at[u]).wait()

  plsc.subcore_barrier()

  pltpu.sync_copy(shared_deg.at[pl.ds(sid * TPN, TPN)],
                  degp_hbm.at[cid].at[pl.ds(sid * TPN, TPN)])

  pltpu.sync_copy(xt_hbm.at[pl.ds(wid, 1)], idxb.at[0])
  for i in range(9):
    pltpu.async_copy(emb_hbm.at[idxb.at[0].at[0].at[i]],
                     rows.at[0].at[pl.ds(i * 128, 128)], sem_g.at[0])

  @pl.loop(wid, NWIN_N, step=2 * NTILES)
  def _(w):
    for u in range(2):
      ww = w + u * NTILES

      @pl.when(ww < NWIN_N)
      def _():
        nxt = ww + NTILES
        v = 1 - u

        @pl.when(nxt < NWIN_N)
        def _():
          pltpu.sync_copy(xt_hbm.at[pl.ds(nxt, 1)], idxb.at[v])
          for i in range(9):
            pltpu.async_copy(emb_hbm.at[idxb.at[v].at[0].at[i]],
                             rows.at[v].at[pl.ds(i * 128, 128)], sem_g.at[v])

        for i in range(9):
          pltpu.make_async_copy(emb_hbm.at[idxb.at[u].at[0].at[i]],
                                rows.at[u].at[pl.ds(i * 128, 128)],
                                sem_g.at[u]).wait()

        @pl.when(ww >= wid + 2 * NTILES)
        def _():
          pltpu.make_async_copy(acc.at[u], h0_hbm.at[pl.ds(0, 128)],
                                sem_w.at[u]).wait()

        @pl.loop(0, 128)
        def _(r):
          for half in range(2):
            cs = pl.ds(half * 16, 16)
            val = rows[u, r, cs]
            for i in range(1, 9):
              val = val + rows[u, i * 128 + r, cs]
            acc[u, r, cs] = val

        pltpu.async_copy(acc.at[u], h0_hbm.at[pl.ds(ww * 128, 128)],
                         sem_w.at[u])

  for u in range(2):
    pltpu.make_async_copy(acc.at[u], h0_hbm.at[pl.ds(0, 128)],
                          sem_w.at[u]).wait()


def _sc_prep(ei3, xt3, emb_flat):
  kfn = pl.kernel(
      _sc_prep_body,
      out_type=(
          jax.ShapeDtypeStruct((NC, NPAD), jnp.float32),
          jax.ShapeDtypeStruct((NODE_PAD, H), jnp.float32),
      ),
      mesh=_mesh,
      scratch_types=[
          pltpu.VMEM((2, SUPD, EW), jnp.int32),
          pltpu.VMEM((EW,), jnp.float32),
          pltpu.VMEM((TPN,), jnp.float32),
          pltpu.VMEM((2, 1, 9, EW), jnp.int32),
          pltpu.VMEM((2, 9 * 128, H), jnp.float32),
          pltpu.VMEM((2, 128, H), jnp.float32),
          pltpu.VMEM_SHARED((NPAD,), jnp.float32),
          pltpu.SemaphoreType.DMA((2,)),
          pltpu.SemaphoreType.DMA((2,)),
          pltpu.SemaphoreType.DMA((2,)),
      ],
      compiler_params=_SC_PARAMS,
  )
  return kfn(ei3, xt3, emb_flat)


def _sc_edge_body(g_hbm, ei_hbm, out_hbm,
                  srcb, dstb, rows, zbuf, shared_acc, sem_g, sem_s):
  cid = lax.axis_index("c")
  sid = lax.axis_index("s")
  wid = sid * NC + cid

  _zero_rows_buf(zbuf)

  zs = []
  for k in range(TPN // ZR):
    zs.append(pltpu.async_copy(
        zbuf, shared_acc.at[pl.ds(sid * TPN + k * ZR, ZR)], sem_g.at[0]))
  for h in zs:
    h.wait()

  plsc.subcore_barrier()

  @pl.loop(wid, NSUP, step=2 * NTILES)
  def _(t):
    for u in range(2):
      tt = t + u * NTILES

      @pl.when(tt < NSUP)
      def _():
        @pl.when(tt >= wid + 2 * NTILES)
        def _():
          for j in range(SUP):
            pltpu.make_async_copy(
                rows.at[u].at[pl.ds(j * 128, 128)],
                shared_acc.at[dstb.at[u].at[j]], sem_s.at[u]).wait()

        pltpu.sync_copy(ei_hbm.at[0].at[pl.ds(tt * SUP, SUP)], srcb.at[u])
        pltpu.sync_copy(ei_hbm.at[1].at[pl.ds(tt * SUP, SUP)], dstb.at[u])

        gs = []
        for j in range(SUP):
          gs.append(pltpu.async_copy(
              g_hbm.at[srcb.at[u].at[j]],
              rows.at[u].at[pl.ds(j * 128, 128)], sem_g.at[u]))
        for h in gs:
          h.wait()

        for j in range(SUP):
          pltpu.async_copy(
              rows.at[u].at[pl.ds(j * 128, 128)],
              shared_acc.at[dstb.at[u].at[j]], sem_s.at[u], add=True)

  for u in range(2):
    for j in range(SUP):
      pltpu.make_async_copy(
          rows.at[u].at[pl.ds(j * 128, 128)],
          shared_acc.at[dstb.at[u].at[j]], sem_s.at[u]).wait()

  plsc.subcore_barrier()

  ds_ = []
  for k in range(TPN // 128):
    sl = pl.ds(sid * TPN + k * 128, 128)
    ds_.append(pltpu.async_copy(
        shared_acc.at[sl], out_hbm.at[cid].at[sl], sem_g.at[0]))
  for h in ds_:
    h.wait()


def _sc_edge(g, ei3):
  kfn = pl.kernel(
      _sc_edge_body,
      out_type=jax.ShapeDtypeStruct((NC, NPAD, H), jnp.float32),
      mesh=_mesh,
      scratch_types=[
          pltpu.VMEM((2, SUP, EW), jnp.int32),
          pltpu.VMEM((2, SUP, EW), jnp.int32),
          pltpu.VMEM((2, SUP * EW, H), jnp.float32),
          pltpu.VMEM((ZR, H), jnp.float32),
          pltpu.VMEM_SHARED((NPAD, H), jnp.float32),
          pltpu.SemaphoreType.DMA((2,)),
          pltpu.SemaphoreType.DMA((2,)),
      ],
      compiler_params=_SC_PARAMS,
  )
  return kfn(g, ei3)


_DOT = functools.partial(
    lax.dot_general,
    precision=lax.Precision.HIGHEST,
    preferred_element_type=jnp.float32,
)


def _mm(a, b):
  return _DOT(a, b, dimension_numbers=(((1,), (0,)), ((), ())))


RB = 2000
GRID_N = N // RB


def _t1_body(h0_ref, dga_ref, dgb_ref, w1_ref, g1_ref, dinv_ref):
  deg = dga_ref[...] + dgb_ref[...] + 1.0
  dinv = lax.rsqrt(deg)
  dinv_ref[...] = dinv
  g1_ref[...] = _mm(h0_ref[...], w1_ref[...]) * dinv


def _t1(h0p, dga, dgb, w1):
  return pl.pallas_call(
      _t1_body,
      grid=(GRID_N,),
      in_specs=[
          pl.BlockSpec((RB, H), lambda i: (i, 0)),
          pl.BlockSpec((RB, 1), lambda i: (i, 0)),
          pl.BlockSpec((RB, 1), lambda i: (i, 0)),
          pl.BlockSpec((H, H), lambda i: (0, 0)),
      ],
      out_specs=[
          pl.BlockSpec((RB, H), lambda i: (i, 0)),
          pl.BlockSpec((RB, 1), lambda i: (i, 0)),
      ],
      out_shape=[
          jax.ShapeDtypeStruct((N, H), jnp.float32),
          jax.ShapeDtypeStruct((N, 1), jnp.float32),
      ],
  )(h0p, dga, dgb, w1)


def _t2_body(s1a_ref, s1b_ref, g1_ref, dinv_ref, b1_ref, w2_ref, g2_ref):
  dinv = dinv_ref[...]
  h1 = dinv * (s1a_ref[0] + s1b_ref[0] + g1_ref[...]) + b1_ref[...]
  h1 = jnp.maximum(h1, 0.0)
  g2_ref[...] = _mm(h1, w2_ref[...]) * dinv


def _t2(s1, g1, dinv, b1, w2):
  return pl.pallas_call(
      _t2_body,
      grid=(GRID_N,),
      in_specs=[
          pl.BlockSpec((1, RB, H), lambda i: (0, i, 0)),
          pl.BlockSpec((1, RB, H), lambda i: (1, i, 0)),
          pl.BlockSpec((RB, H), lambda i: (i, 0)),
          pl.BlockSpec((RB, 1), lambda i: (i, 0)),
          pl.BlockSpec((1, H), lambda i: (0, 0)),
          pl.BlockSpec((H, H), lambda i: (0, 0)),
      ],
      out_specs=pl.BlockSpec((RB, H), lambda i: (i, 0)),
      out_shape=jax.ShapeDtypeStruct((N, H), jnp.float32),
  )(s1, s1, g1, dinv, b1, w2)


def _t3_body(s2a_ref, s2b_ref, g2_ref, dinv_ref, b2_ref, batch_ref,
             wout_ref, bout_ref, out_ref, sums_ref, cnt_ref):
  i = pl.program_id(0)

  @pl.when(i == 0)
  def _():
    sums_ref[...] = jnp.zeros_like(sums_ref)
    cnt_ref[...] = jnp.zeros_like(cnt_ref)

  dinv = dinv_ref[...]
  h2 = dinv * (s2a_ref[0] + s2b_ref[0] + g2_ref[...]) + b2_ref[...]
  seg = lax.broadcasted_iota(jnp.int32, (B, RB), 0)
  mask = (seg == batch_ref[...][0]).astype(jnp.float32)
  sums_ref[...] += _mm(mask, h2)
  cnt_ref[...] += jnp.sum(mask, axis=1, keepdims=True)

  @pl.when(i == GRID_N - 1)
  def _():
    pooled = sums_ref[...] / jnp.maximum(cnt_ref[...], 1.0)
    out_ref[...] = _mm(pooled, wout_ref[...]) + bout_ref[...]


def _t3(s2, g2, dinv, b2, batch3d, wout, bout):
  return pl.pallas_call(
      _t3_body,
      grid=(GRID_N,),
      in_specs=[
          pl.BlockSpec((1, RB, H), lambda i: (0, i, 0)),
          pl.BlockSpec((1, RB, H), lambda i: (1, i, 0)),
          pl.BlockSpec((RB, H), lambda i: (i, 0)),
          pl.BlockSpec((RB, 1), lambda i: (i, 0)),
          pl.BlockSpec((1, H), lambda i: (0, 0)),
          pl.BlockSpec((1, 1, RB), lambda i: (i, 0, 0)),
          pl.BlockSpec((H, NUM_CLASSES), lambda i: (0, 0)),
          pl.BlockSpec((1, NUM_CLASSES), lambda i: (0, 0)),
      ],
      out_specs=pl.BlockSpec((B, NUM_CLASSES), lambda i: (0, 0)),
      out_shape=jax.ShapeDtypeStruct((B, NUM_CLASSES), jnp.float32),
      scratch_shapes=[
          pltpu.VMEM((B, H), jnp.float32),
          pltpu.VMEM((B, 1), jnp.float32),
      ],
  )(s2, s2, g2, dinv, b2, batch3d, wout, bout)


@jax.jit
def kernel(x, edge_index, batch, emb, W1, b1, W2, b2, Wout, bout):
  x = x.astype(jnp.int32)
  edge_index = edge_index.astype(jnp.int32)
  batch = batch.astype(jnp.int32)

  xi = x + (jnp.arange(9, dtype=jnp.int32) * 100)[None, :]
  xt = jnp.zeros((9, NODE_PAD), jnp.int32).at[:, :N].set(xi.T)
  xt3 = xt.reshape(9, NWIN_N, 128).transpose(1, 0, 2)
  emb_flat = emb.reshape(9 * 100, H)

  ei3 = edge_index.reshape(2, NWIN_E, EW)

  degp, h0p = _sc_prep(ei3, xt3, emb_flat)
  dga = degp[0, :N, None]
  dgb = degp[1, :N, None]

  g1, dinv = _t1(h0p, dga, dgb, W1)

  s1 = _sc_edge(g1, ei3)
  g2 = _t2(s1, g1, dinv, b1.reshape(1, H), W2)

  s2 = _sc_edge(g2, ei3)
  out = _t3(s2, g2, dinv, b2.reshape(1, H),
            batch.reshape(GRID_N, 1, RB), Wout, bout.reshape(1, NUM_CLASSES))
  return out


# speedup vs baseline: 42.1115x; 1.3707x over previous
import functools

import jax
import jax.numpy as jnp
from jax import lax
from jax.experimental import pallas as pl
from jax.experimental.pallas import tpu as pltpu
from jax.experimental.pallas import tpu_sc as plsc

N = 50000
E = 1600000
H = 32
NUM_CLASSES = 128
B = 32

NC = 2
NS = 16
NTILES = NC * NS
EW = 128

NPAD = 50048
TPN = NPAD // NS
NWIN_N = NPAD // 128

NWIN_EP = 12544
E_PAD = NWIN_EP * EW
SUP = 2
NSUP = NWIN_EP // SUP
KPT = NSUP // NTILES
KPT3 = (KPT + 2) // 3
SUPD = 7
NSUPD = NWIN_EP // SUPD
DPT = NSUPD // NTILES

ZR = 68
NZC = TPN // ZR

_mesh = plsc.VectorSubcoreMesh(core_axis_name="c", subcore_axis_name="s")
_SC_PARAMS = pltpu.CompilerParams(use_tc_tiling_on_sc=False)


def _zero_rows_buf(buf):
  z16 = jnp.zeros((16,), jnp.float32)

  @pl.loop(0, ZR)
  def _(r):
    buf[r, pl.ds(0, 16)] = z16
    buf[r, pl.ds(16, 16)] = z16


def _sc_prep_body(ei_hbm, xt_hbm, emb_hbm, degp_hbm, h0_hbm,
                  dstb, onesb, zdeg, idxb, rows, acc, shared_deg,
                  sem_d, sem_g, sem_w):
  cid = lax.axis_index("c")
  sid = lax.axis_index("s")
  wid = sid * NC + cid

  @pl.loop(0, (TPN + 15) // 16)
  def _(k):
    zdeg[pl.ds(k * 16, 16)] = jnp.zeros((16,), jnp.float32)

  pltpu.sync_copy(zdeg.at[pl.ds(0, TPN)], shared_deg.at[pl.ds(sid * TPN, TPN)])

  for k in range(8):
    onesb[pl.ds(k * 16, 16)] = jnp.full((16,), 1.0, jnp.float32)

  plsc.subcore_barrier()

  @pl.loop(0, DPT, step=2)
  def _(jj):
    for u in range(2):
      m = jj + u
      tt = wid + m * NTILES

      @pl.when(m >= 2)
      def _():
        for j in range(SUPD):
          pltpu.make_async_copy(
              onesb, shared_deg.at[dstb.at[u].at[j]], sem_d.at[u]).wait()

      pltpu.sync_copy(ei_hbm.at[1].at[pl.ds(tt * SUPD, SUPD)], dstb.at[u])
      for j in range(SUPD):
        pltpu.async_copy(
            onesb, shared_deg.at[dstb.at[u].at[j]], sem_d.at[u], add=True)

  for u in range(2):
    for j in range(SUPD):
      pltpu.make_async_copy(
          onesb, shared_deg.at[dstb.at[u].at[j]], sem_d.at[u]).wait()

  plsc.subcore_barrier()

  pltpu.sync_copy(shared_deg.at[pl.ds(sid * TPN, TPN)],
                  degp_hbm.at[cid].at[pl.ds(sid * TPN, TPN)])

  pltpu.sync_copy(xt_hbm.at[pl.ds(wid, 1)], idxb.at[0])
  for i in range(9):
    pltpu.async_copy(emb_hbm.at[idxb.at[0].at[0].at[i]],
                     rows.at[0].at[pl.ds(i * 128, 128)], sem_g.at[0])

  @pl.loop(wid, NWIN_N, step=2 * NTILES)
  def _(w):
    for u in range(2):
      ww = w + u * NTILES

      @pl.when(ww < NWIN_N)
      def _():
        nxt = ww + NTILES
        v = 1 - u

        @pl.when(nxt < NWIN_N)
        def _():
          pltpu.sync_copy(xt_hbm.at[pl.ds(nxt, 1)], idxb.at[v])
          for i in range(9):
            pltpu.async_copy(emb_hbm.at[idxb.at[v].at[0].at[i]],
                             rows.at[v].at[pl.ds(i * 128, 128)], sem_g.at[v])

        for i in range(9):
          pltpu.make_async_copy(emb_hbm.at[idxb.at[u].at[0].at[i]],
                                rows.at[u].at[pl.ds(i * 128, 128)],
                                sem_g.at[u]).wait()

        @pl.when(ww >= wid + 2 * NTILES)
        def _():
          pltpu.make_async_copy(acc.at[u], h0_hbm.at[pl.ds(0, 128)],
                                sem_w.at[u]).wait()

        @pl.loop(0, 128)
        def _(r):
          for half in range(2):
            cs = pl.ds(half * 16, 16)
            val = rows[u, r, cs]
            for i in range(1, 9):
              val = val + rows[u, i * 128 + r, cs]
            acc[u, r, cs] = val

        pltpu.async_copy(acc.at[u], h0_hbm.at[pl.ds(ww * 128, 128)],
                         sem_w.at[u])

  for u in range(2):
    pltpu.make_async_copy(acc.at[u], h0_hbm.at[pl.ds(0, 128)],
                          sem_w.at[u]).wait()


def _sc_prep(ei3, xt3, emb_flat):
  kfn = pl.kernel(
      _sc_prep_body,
      out_type=(
          jax.ShapeDtypeStruct((NC, NPAD), jnp.float32),
          jax.ShapeDtypeStruct((NPAD, H), jnp.float32),
      ),
      mesh=_mesh,
      scratch_types=[
          pltpu.VMEM((2, SUPD, EW), jnp.int32),
          pltpu.VMEM((EW,), jnp.float32),
          pltpu.VMEM((TPN + 8,), jnp.float32),
          pltpu.VMEM((2, 1, 9, EW), jnp.int32),
          pltpu.VMEM((2, 9 * 128, H), jnp.float32),
          pltpu.VMEM((2, 128, H), jnp.float32),
          pltpu.VMEM_SHARED((NPAD,), jnp.float32),
          pltpu.SemaphoreType.DMA((2,)),
          pltpu.SemaphoreType.DMA((2,)),
          pltpu.SemaphoreType.DMA((2,)),
      ],
      compiler_params=_SC_PARAMS,
  )
  return kfn(ei3, xt3, emb_flat)


def _sc_edge_body(g_hbm, ei_hbm, out_hbm,
                  sdb, rows, zbuf, shared_acc, sem_g, sem_s):
  cid = lax.axis_index("c")
  sid = lax.axis_index("s")
  wid = sid * NC + cid

  _zero_rows_buf(zbuf)

  zs = []
  for k in range(NZC):
    zs.append(pltpu.async_copy(
        zbuf, shared_acc.at[pl.ds(sid * TPN + k * ZR, ZR)], sem_g.at[0]))
  for h in zs:
    h.wait()

  plsc.subcore_barrier()

  def load_idx(b, tt):
    pltpu.sync_copy(ei_hbm.at[:, pl.ds(tt * SUP, SUP)], sdb.at[b])

  def fire_gathers(b):
    for j in range(SUP):
      pltpu.async_copy(g_hbm.at[sdb.at[b].at[0].at[j]],
                       rows.at[b].at[pl.ds(j * 128, 128)], sem_g.at[b])

  def wait_gathers(b):
    for j in range(SUP):
      pltpu.make_async_copy(g_hbm.at[sdb.at[b].at[0].at[j]],
                            rows.at[b].at[pl.ds(j * 128, 128)],
                            sem_g.at[b]).wait()

  def fire_scatters(b):
    for j in range(SUP):
      pltpu.async_copy(rows.at[b].at[pl.ds(j * 128, 128)],
                       shared_acc.at[sdb.at[b].at[1].at[j]],
                       sem_s.at[b], add=True)

  def wait_scatters(b):
    for j in range(SUP):
      pltpu.make_async_copy(rows.at[b].at[pl.ds(j * 128, 128)],
                            shared_acc.at[sdb.at[b].at[1].at[j]],
                            sem_s.at[b]).wait()

  load_idx(0, wid)
  fire_gathers(0)

  @pl.loop(0, KPT3)
  def _(q):
    for slot in range(3):
      k = 3 * q + slot

      @pl.when(k < KPT)
      def _():
        k1 = k + 1
        u1 = (slot + 1) % 3

        @pl.when(k1 < KPT)
        def _():
          @pl.when(k1 >= 3)
          def _():
            wait_scatters(u1)

          load_idx(u1, wid + k1 * NTILES)
          fire_gathers(u1)

        wait_gathers(slot)
        fire_scatters(slot)

  wait_scatters((KPT - 3) % 3)
  wait_scatters((KPT - 2) % 3)
  wait_scatters((KPT - 1) % 3)

  plsc.subcore_barrier()

  ds_ = []
  for k in range(NZC):
    sl = pl.ds(sid * TPN + k * ZR, ZR)
    ds_.append(pltpu.async_copy(
        shared_acc.at[sl], out_hbm.at[cid].at[sl], sem_g.at[0]))
  for h in ds_:
    h.wait()


def _sc_edge(g, ei3):
  kfn = pl.kernel(
      _sc_edge_body,
      out_type=jax.ShapeDtypeStruct((NC, NPAD, H), jnp.float32),
      mesh=_mesh,
      scratch_types=[
          pltpu.VMEM((3, 2, SUP, EW), jnp.int32),
          pltpu.VMEM((3, SUP * EW, H), jnp.float32),
          pltpu.VMEM((ZR, H), jnp.float32),
          pltpu.VMEM_SHARED((NPAD, H), jnp.float32),
          pltpu.SemaphoreType.DMA((3,)),
          pltpu.SemaphoreType.DMA((3,)),
      ],
      compiler_params=_SC_PARAMS,
  )
  return kfn(g, ei3)


_DOT = functools.partial(
    lax.dot_general,
    precision=lax.Precision.HIGHEST,
    preferred_element_type=jnp.float32,
)


def _mm(a, b):
  return _DOT(a, b, dimension_numbers=(((1,), (0,)), ((), ())))


RB = 2000
GRID_N = N // RB


def _t1_body(h0_ref, dga_ref, dgb_ref, w1_ref, g1_ref, dinv_ref):
  deg = dga_ref[...] + dgb_ref[...] + 1.0
  dinv = lax.rsqrt(deg)
  dinv_ref[...] = dinv
  g1_ref[...] = _mm(h0_ref[...], w1_ref[...]) * dinv


def _t1(h0p, dga, dgb, w1):
  return pl.pallas_call(
      _t1_body,
      grid=(GRID_N,),
      in_specs=[
          pl.BlockSpec((RB, H), lambda i: (i, 0)),
          pl.BlockSpec((RB, 1), lambda i: (i, 0)),
          pl.BlockSpec((RB, 1), lambda i: (i, 0)),
          pl.BlockSpec((H, H), lambda i: (0, 0)),
      ],
      out_specs=[
          pl.BlockSpec((RB, H), lambda i: (i, 0)),
          pl.BlockSpec((RB, 1), lambda i: (i, 0)),
      ],
      out_shape=[
          jax.ShapeDtypeStruct((N, H), jnp.float32),
          jax.ShapeDtypeStruct((N, 1), jnp.float32),
      ],
  )(h0p, dga, dgb, w1)


def _t2_body(s1a_ref, s1b_ref, g1_ref, dinv_ref, b1_ref, w2_ref, g2_ref):
  dinv = dinv_ref[...]
  h1 = dinv * (s1a_ref[0] + s1b_ref[0] + g1_ref[...]) + b1_ref[...]
  h1 = jnp.maximum(h1, 0.0)
  g2_ref[...] = _mm(h1, w2_ref[...]) * dinv


def _t2(s1, g1, dinv, b1, w2):
  return pl.pallas_call(
      _t2_body,
      grid=(GRID_N,),
      in_specs=[
          pl.BlockSpec((1, RB, H), lambda i: (0, i, 0)),
          pl.BlockSpec((1, RB, H), lambda i: (1, i, 0)),
          pl.BlockSpec((RB, H), lambda i: (i, 0)),
          pl.BlockSpec((RB, 1), lambda i: (i, 0)),
          pl.BlockSpec((1, H), lambda i: (0, 0)),
          pl.BlockSpec((H, H), lambda i: (0, 0)),
      ],
      out_specs=pl.BlockSpec((RB, H), lambda i: (i, 0)),
      out_shape=jax.ShapeDtypeStruct((N, H), jnp.float32),
  )(s1, s1, g1, dinv, b1, w2)


def _t3_body(s2a_ref, s2b_ref, g2_ref, dinv_ref, b2_ref, batch_ref,
             wout_ref, bout_ref, out_ref, sums_ref, cnt_ref):
  i = pl.program_id(0)

  @pl.when(i == 0)
  def _():
    sums_ref[...] = jnp.zeros_like(sums_ref)
    cnt_ref[...] = jnp.zeros_like(cnt_ref)

  dinv = dinv_ref[...]
  h2 = dinv * (s2a_ref[0] + s2b_ref[0] + g2_ref[...]) + b2_ref[...]
  seg = lax.broadcasted_iota(jnp.int32, (B, RB), 0)
  mask = (seg == batch_ref[...][0]).astype(jnp.float32)
  sums_ref[...] += _mm(mask, h2)
  cnt_ref[...] += jnp.sum(mask, axis=1, keepdims=True)

  @pl.when(i == GRID_N - 1)
  def _():
    pooled = sums_ref[...] / jnp.maximum(cnt_ref[...], 1.0)
    out_ref[...] = _mm(pooled, wout_ref[...]) + bout_ref[...]


def _t3(s2, g2, dinv, b2, batch3d, wout, bout):
  return pl.pallas_call(
      _t3_body,
      grid=(GRID_N,),
      in_specs=[
          pl.BlockSpec((1, RB, H), lambda i: (0, i, 0)),
          pl.BlockSpec((1, RB, H), lambda i: (1, i, 0)),
          pl.BlockSpec((RB, H), lambda i: (i, 0)),
          pl.BlockSpec((RB, 1), lambda i: (i, 0)),
          pl.BlockSpec((1, H), lambda i: (0, 0)),
          pl.BlockSpec((1, 1, RB), lambda i: (i, 0, 0)),
          pl.BlockSpec((H, NUM_CLASSES), lambda i: (0, 0)),
          pl.BlockSpec((1, NUM_CLASSES), lambda i: (0, 0)),
      ],
      out_specs=pl.BlockSpec((B, NUM_CLASSES), lambda i: (0, 0)),
      out_shape=jax.ShapeDtypeStruct((B, NUM_CLASSES), jnp.float32),
      scratch_shapes=[
          pltpu.VMEM((B, H), jnp.float32),
          pltpu.VMEM((B, 1), jnp.float32),
      ],
  )(s2, s2, g2, dinv, b2, batch3d, wout, bout)


@jax.jit
def kernel(x, edge_index, batch, emb, W1, b1, W2, b2, Wout, bout):
  x = x.astype(jnp.int32)
  edge_index = edge_index.astype(jnp.int32)
  batch = batch.astype(jnp.int32)

  xi = x + (jnp.arange(9, dtype=jnp.int32) * 100)[None, :]
  xt = jnp.zeros((9, NPAD), jnp.int32).at[:, :N].set(xi.T)
  xt3 = xt.reshape(9, NWIN_N, 128).transpose(1, 0, 2)
  emb_flat = emb.reshape(9 * 100, H)

  pad = jnp.broadcast_to(
      jnp.array([[0], [NPAD - 1]], jnp.int32), (2, E_PAD - E))
  ei3 = jnp.concatenate([edge_index, pad], axis=1).reshape(2, NWIN_EP, EW)

  degp, h0p = _sc_prep(ei3, xt3, emb_flat)
  dga = degp[0, :N, None]
  dgb = degp[1, :N, None]

  g1, dinv = _t1(h0p, dga, dgb, W1)

  s1 = _sc_edge(g1, ei3)
  g2 = _t2(s1, g1, dinv, b1.reshape(1, H), W2)

  s2 = _sc_edge(g2, ei3)
  out = _t3(s2, g2, dinv, b2.reshape(1, H),
            batch.reshape(GRID_N, 1, RB), Wout, bout.reshape(1, NUM_CLASSES))
  return out
